# baseline (device time: 128018 ns/iter reference)
import jax
import jax.numpy as jnp
from jax import lax
from jax.experimental import pallas as pl
from jax.experimental.pallas import tpu as pltpu

N_DEV = 16
M = 1024
D = 1024
CHUNK = M // N_DEV
N_HOPS = N_DEV - 1


def kernel(x, Wg, Wu, Wd):
    x = x.astype(jnp.bfloat16)
    Wg = Wg.astype(jnp.bfloat16)
    Wu = Wu.astype(jnp.bfloat16)
    Wd = Wd.astype(jnp.bfloat16)

    def body(x_ref, wg_ref, wu_ref, wd_ref, out_ref,
             acc_ref, comm_ref, stage_ref, send_sem, recv_sems):
        my = lax.axis_index("i")
        left = (my - 1) % N_DEV
        right = (my + 1) % N_DEV

        barrier_sem = pltpu.get_barrier_semaphore()
        for nbr in (left, right):
            pl.semaphore_signal(
                barrier_sem, inc=1,
                device_id=(nbr,), device_id_type=pl.DeviceIdType.MESH,
            )
        pl.semaphore_wait(barrier_sem, 2)

        xb = x_ref[:, :]
        gate = jnp.dot(xb, wg_ref[:, :], preferred_element_type=jnp.float32)
        up = jnp.dot(xb, wu_ref[:, :], preferred_element_type=jnp.float32)
        h = (gate * (up * jax.nn.sigmoid(up))).astype(jnp.bfloat16)
        acc_ref[:, :] = jnp.dot(
            h, wd_ref[:, :], preferred_element_type=jnp.float32
        )

        def hop(slot, send_from, c_recv_store):
            rdma = pltpu.make_async_remote_copy(
                src_ref=send_from,
                dst_ref=comm_ref.at[slot],
                send_sem=send_sem,
                recv_sem=recv_sems.at[slot],
                device_id=(right,),
                device_id_type=pl.DeviceIdType.MESH,
            )
            rdma.start()
            rdma.wait()

        for h in range(N_HOPS):
            c_send = (my - h) % N_DEV
            c_recv = (my - h - 1) % N_DEV
            stage_ref[:, :] = acc_ref[pl.ds(c_send * CHUNK, CHUNK), :].astype(
                jnp.bfloat16
            )
            hop(h, stage_ref, None)
            acc_ref[pl.ds(c_recv * CHUNK, CHUNK), :] = (
                acc_ref[pl.ds(c_recv * CHUNK, CHUNK), :]
                + comm_ref[h].astype(jnp.float32)
            )

        own = (my + 1) % N_DEV
        out_ref[pl.ds(own * CHUNK, CHUNK), :] = acc_ref[
            pl.ds(own * CHUNK, CHUNK), :
        ]

        for h in range(N_HOPS):
            slot = N_HOPS + h
            if h == 0:
                stage_ref[:, :] = acc_ref[pl.ds(own * CHUNK, CHUNK), :].astype(
                    jnp.bfloat16
                )
                src = stage_ref
            else:
                src = comm_ref.at[slot - 1]
            hop(slot, src, None)
            c_recv = (my - h) % N_DEV
            out_ref[pl.ds(c_recv * CHUNK, CHUNK), :] = comm_ref[slot].astype(
                jnp.float32
            )

    return pl.pallas_call(
        body,
        out_shape=jax.ShapeDtypeStruct((M, D), jnp.float32),
        in_specs=[pl.BlockSpec(memory_space=pltpu.VMEM)] * 4,
        out_specs=pl.BlockSpec(memory_space=pltpu.VMEM),
        scratch_shapes=[
            pltpu.VMEM((M, D), jnp.float32),
            pltpu.VMEM((2 * N_HOPS, CHUNK, D), jnp.bfloat16),
            pltpu.VMEM((CHUNK, D), jnp.bfloat16),
            pltpu.SemaphoreType.DMA,
            pltpu.SemaphoreType.DMA((2 * N_HOPS,)),
        ],
        compiler_params=pltpu.CompilerParams(collective_id=0),
    )(x, Wg, Wu, Wd)


# device time: 89812 ns/iter; 1.4254x vs baseline; 1.4254x over previous
import jax
import jax.numpy as jnp
from jax import lax
from jax.experimental import pallas as pl
from jax.experimental.pallas import tpu as pltpu

N_DEV = 16
M = 1024
D = 1024
LOG_N = 4

_RS_OFF = [0, 512, 768, 896]
_COMM_ROWS = 960


def kernel(x, Wg, Wu, Wd):
    x = x.astype(jnp.bfloat16)
    Wg = Wg.astype(jnp.bfloat16)
    Wu = Wu.astype(jnp.bfloat16)
    Wd = Wd.astype(jnp.bfloat16)

    def body(x_ref, wg_ref, wu_ref, wd_ref, out_ref,
             acc_ref, comm_ref, stage_ref, send_sem, recv_sems):
        my = lax.axis_index("i")
        partners = [my ^ (1 << k) for k in range(LOG_N)]

        barrier_sem = pltpu.get_barrier_semaphore()
        for p in partners:
            pl.semaphore_signal(
                barrier_sem, inc=1,
                device_id=(p,), device_id_type=pl.DeviceIdType.MESH,
            )
        pl.semaphore_wait(barrier_sem, LOG_N)

        xb = x_ref[:, :]
        gate = jnp.dot(xb, wg_ref[:, :], preferred_element_type=jnp.float32)
        up = jnp.dot(xb, wu_ref[:, :], preferred_element_type=jnp.float32)
        h = (gate * (up * jax.nn.sigmoid(up))).astype(jnp.bfloat16)
        acc_ref[:, :] = jnp.dot(
            h, wd_ref[:, :], preferred_element_type=jnp.float32
        )

        s = my * 0
        for k in range(LOG_N):
            d = 1 << k
            half = 512 >> k
            partner = my ^ d
            upper = (my & d) != 0
            send_start = pl.multiple_of(s + jnp.where(upper, 0, half), 64)
            keep_start = pl.multiple_of(s + jnp.where(upper, half, 0), 64)
            stage_ref[pl.ds(0, half), :] = acc_ref[
                pl.ds(send_start, half), :
            ].astype(jnp.bfloat16)
            rdma = pltpu.make_async_remote_copy(
                src_ref=stage_ref.at[pl.ds(0, half), :],
                dst_ref=comm_ref.at[pl.ds(_RS_OFF[k], half), :],
                send_sem=send_sem,
                recv_sem=recv_sems.at[k],
                device_id=(partner,),
                device_id_type=pl.DeviceIdType.MESH,
            )
            rdma.start()
            rdma.wait()
            acc_ref[pl.ds(keep_start, half), :] = (
                acc_ref[pl.ds(keep_start, half), :]
                + comm_ref[pl.ds(_RS_OFF[k], half), :].astype(jnp.float32)
            )
            s = keep_start

        out_ref[pl.ds(s, 64), :] = acc_ref[pl.ds(s, 64), :].astype(
            jnp.bfloat16
        )

        for k in range(LOG_N):
            d = 8 >> k
            L = 64 << k
            partner = my ^ d
            s = pl.multiple_of(s, 64)
            rdma = pltpu.make_async_remote_copy(
                src_ref=out_ref.at[pl.ds(s, L), :],
                dst_ref=out_ref.at[pl.ds(s, L), :],
                send_sem=send_sem,
                recv_sem=recv_sems.at[LOG_N + k],
                device_id=(partner,),
                device_id_type=pl.DeviceIdType.MESH,
            )
            rdma.start()
            rdma.wait()
            s = jnp.where((my & d) != 0, s - L, s)

    return pl.pallas_call(
        body,
        out_shape=jax.ShapeDtypeStruct((M, D), jnp.bfloat16),
        in_specs=[pl.BlockSpec(memory_space=pltpu.VMEM)] * 4,
        out_specs=pl.BlockSpec(memory_space=pltpu.VMEM),
        scratch_shapes=[
            pltpu.VMEM((M, D), jnp.float32),
            pltpu.VMEM((_COMM_ROWS, D), jnp.bfloat16),
            pltpu.VMEM((512, D), jnp.bfloat16),
            pltpu.SemaphoreType.DMA,
            pltpu.SemaphoreType.DMA((2 * LOG_N,)),
        ],
        compiler_params=pltpu.CompilerParams(collective_id=0),
    )(x, Wg, Wu, Wd)


# device time: 82295 ns/iter; 1.5556x vs baseline; 1.0913x over previous
import jax
import jax.numpy as jnp
from jax import lax
from jax.experimental import pallas as pl
from jax.experimental.pallas import tpu as pltpu

N_DEV = 16
M = 1024
D = 1024
LOG_N = 4

_RS_OFF = [0, 512, 768, 896]
_COMM_ROWS = 960


def kernel(x, Wg, Wu, Wd):
    x = x.astype(jnp.bfloat16)
    Wg = Wg.astype(jnp.bfloat16)
    Wu = Wu.astype(jnp.bfloat16)
    Wd = Wd.astype(jnp.bfloat16)

    def body(x_ref, wg_ref, wu_ref, wd_ref, out_ref,
             acc_ref, comm_ref, stage_ref, send_sem, recv_sems):
        my = lax.axis_index("i")
        partners = [my ^ (1 << k) for k in range(LOG_N)]

        barrier_sem = pltpu.get_barrier_semaphore()
        for p in partners:
            pl.semaphore_signal(
                barrier_sem, inc=1,
                device_id=(p,), device_id_type=pl.DeviceIdType.MESH,
            )
        pl.semaphore_wait(barrier_sem, LOG_N)

        def compute_half(start):
            xh = x_ref[pl.ds(start, 512), :]
            gate = jnp.dot(xh, wg_ref[:, :], preferred_element_type=jnp.float32)
            up = jnp.dot(xh, wu_ref[:, :], preferred_element_type=jnp.float32)
            h = (gate * (up * jax.nn.sigmoid(up))).astype(jnp.bfloat16)
            acc_ref[pl.ds(start, 512), :] = jnp.dot(
                h, wd_ref[:, :], preferred_element_type=jnp.float32
            )

        s = my * 0
        for k in range(LOG_N):
            d = 1 << k
            half = 512 >> k
            partner = my ^ d
            upper = (my & d) != 0
            send_start = pl.multiple_of(s + jnp.where(upper, 0, half), 64)
            keep_start = pl.multiple_of(s + jnp.where(upper, half, 0), 64)
            if k == 0:
                compute_half(send_start)
            stage_ref[pl.ds(0, half), :] = acc_ref[
                pl.ds(send_start, half), :
            ].astype(jnp.bfloat16)
            rdma = pltpu.make_async_remote_copy(
                src_ref=stage_ref.at[pl.ds(0, half), :],
                dst_ref=comm_ref.at[pl.ds(_RS_OFF[k], half), :],
                send_sem=send_sem,
                recv_sem=recv_sems.at[k],
                device_id=(partner,),
                device_id_type=pl.DeviceIdType.MESH,
            )
            rdma.start()
            if k == 0:
                compute_half(keep_start)
            rdma.wait()
            acc_ref[pl.ds(keep_start, half), :] = (
                acc_ref[pl.ds(keep_start, half), :]
                + comm_ref[pl.ds(_RS_OFF[k], half), :].astype(jnp.float32)
            )
            s = keep_start

        out_ref[pl.ds(s, 64), :] = acc_ref[pl.ds(s, 64), :].astype(
            jnp.bfloat16
        )

        for k in range(LOG_N):
            d = 8 >> k
            L = 64 << k
            partner = my ^ d
            s = pl.multiple_of(s, 64)
            rdma = pltpu.make_async_remote_copy(
                src_ref=out_ref.at[pl.ds(s, L), :],
                dst_ref=out_ref.at[pl.ds(s, L), :],
                send_sem=send_sem,
                recv_sem=recv_sems.at[LOG_N + k],
                device_id=(partner,),
                device_id_type=pl.DeviceIdType.MESH,
            )
            rdma.start()
            rdma.wait()
            s = jnp.where((my & d) != 0, s - L, s)

    return pl.pallas_call(
        body,
        out_shape=jax.ShapeDtypeStruct((M, D), jnp.bfloat16),
        in_specs=[pl.BlockSpec(memory_space=pltpu.VMEM)] * 4,
        out_specs=pl.BlockSpec(memory_space=pltpu.VMEM),
        scratch_shapes=[
            pltpu.VMEM((M, D), jnp.float32),
            pltpu.VMEM((_COMM_ROWS, D), jnp.bfloat16),
            pltpu.VMEM((512, D), jnp.bfloat16),
            pltpu.SemaphoreType.DMA,
            pltpu.SemaphoreType.DMA((2 * LOG_N,)),
        ],
        compiler_params=pltpu.CompilerParams(collective_id=0),
    )(x, Wg, Wu, Wd)


# device time: 74661 ns/iter; 1.7147x vs baseline; 1.1022x over previous
import jax
import jax.numpy as jnp
from jax import lax
from jax.experimental import pallas as pl
from jax.experimental.pallas import tpu as pltpu

N_DEV = 16
M = 1024
D = 1024
HC = D // 2
LOG_N = 4

_RS_DIMS_A = (1, 2, 4, 8)
_RS_DIMS_B = (4, 8, 1, 2)
_AG_DIMS_A = (8, 4, 2, 1)
_AG_DIMS_B = (2, 1, 8, 4)
_RS_OFF = (0, 512, 768, 896)
_COMM_ROWS = 960


def kernel(x, Wg, Wu, Wd):
    x = x.astype(jnp.bfloat16)
    Wg = Wg.astype(jnp.bfloat16)
    Wu = Wu.astype(jnp.bfloat16)
    Wd = Wd.astype(jnp.bfloat16)

    def body(x_ref, wg_ref, wu_ref, wd_ref, out_ref,
             acc_ref, comm_ref, stage_ref, send_sems, recv_sems):
        my = lax.axis_index("i")

        barrier_sem = pltpu.get_barrier_semaphore()
        for k in range(LOG_N):
            pl.semaphore_signal(
                barrier_sem, inc=1,
                device_id=(my ^ (1 << k),),
                device_id_type=pl.DeviceIdType.MESH,
            )
        pl.semaphore_wait(barrier_sem, LOG_N)

        def compute_half(start):
            xh = x_ref[pl.ds(start, 512), :]
            gate = jnp.dot(xh, wg_ref[:, :], preferred_element_type=jnp.float32)
            up = jnp.dot(xh, wu_ref[:, :], preferred_element_type=jnp.float32)
            h = (gate * (up * jax.nn.sigmoid(up))).astype(jnp.bfloat16)
            acc_ref[pl.ds(start, 512), :] = jnp.dot(
                h, wd_ref[:, :], preferred_element_type=jnp.float32
            )

        def rs_exchange(t, half, send_start, cbase, snd, sem_idx, partner):
            cols = pl.ds(cbase, HC)
            stage_ref[pl.ds(0, half), cols] = acc_ref[
                pl.ds(send_start, half), cols
            ].astype(jnp.bfloat16)
            rdma = pltpu.make_async_remote_copy(
                src_ref=stage_ref.at[pl.ds(0, half), cols],
                dst_ref=comm_ref.at[pl.ds(_RS_OFF[t], half), cols],
                send_sem=send_sems.at[snd],
                recv_sem=recv_sems.at[sem_idx],
                device_id=(partner,),
                device_id_type=pl.DeviceIdType.MESH,
            )
            rdma.start()
            return rdma

        sA = my * 0
        sB = my * 0
        for t in range(LOG_N):
            half = 512 >> t
            dA, dB = _RS_DIMS_A[t], _RS_DIMS_B[t]
            upA, upB = (my & dA) != 0, (my & dB) != 0
            sendA = pl.multiple_of(sA + jnp.where(upA, 0, half), 64)
            keepA = pl.multiple_of(sA + jnp.where(upA, half, 0), 64)
            sendB = pl.multiple_of(sB + jnp.where(upB, 0, half), 64)
            keepB = pl.multiple_of(sB + jnp.where(upB, half, 0), 64)
            if t == 0:
                compute_half(sendA)
            rdma_a = rs_exchange(t, half, sendA, 0, 0, t, my ^ dA)
            if t == 0:
                compute_half(keepA)
            rdma_b = rs_exchange(t, half, sendB, HC, 1, LOG_N + t, my ^ dB)
            rdma_a.wait()
            rdma_b.wait()
            for keep, cbase, off in ((keepA, 0, 0), (keepB, HC, 1)):
                cols = pl.ds(cbase, HC)
                acc_ref[pl.ds(keep, half), cols] = (
                    acc_ref[pl.ds(keep, half), cols]
                    + comm_ref[pl.ds(_RS_OFF[t], half), cols].astype(
                        jnp.float32
                    )
                )
            sA, sB = keepA, keepB

        sA = pl.multiple_of(sA, 64)
        sB = pl.multiple_of(sB, 64)
        out_ref[pl.ds(sA, 64), pl.ds(0, HC)] = acc_ref[
            pl.ds(sA, 64), pl.ds(0, HC)
        ].astype(jnp.bfloat16)
        out_ref[pl.ds(sB, 64), pl.ds(HC, HC)] = acc_ref[
            pl.ds(sB, 64), pl.ds(HC, HC)
        ].astype(jnp.bfloat16)

        def ag_exchange(L, s, cbase, snd, sem_idx, partner):
            cols = pl.ds(cbase, HC)
            rdma = pltpu.make_async_remote_copy(
                src_ref=out_ref.at[pl.ds(s, L), cols],
                dst_ref=out_ref.at[pl.ds(s, L), cols],
                send_sem=send_sems.at[snd],
                recv_sem=recv_sems.at[sem_idx],
                device_id=(partner,),
                device_id_type=pl.DeviceIdType.MESH,
            )
            rdma.start()
            return rdma

        for t in range(LOG_N):
            L = 64 << t
            dA, dB = _AG_DIMS_A[t], _AG_DIMS_B[t]
            rdma_a = ag_exchange(L, sA, 0, 0, 2 * LOG_N + t, my ^ dA)
            rdma_b = ag_exchange(L, sB, HC, 1, 3 * LOG_N + t, my ^ dB)
            rdma_a.wait()
            rdma_b.wait()
            sA = pl.multiple_of(jnp.where((my & dA) != 0, sA - L, sA), 64)
            sB = pl.multiple_of(jnp.where((my & dB) != 0, sB - L, sB), 64)

    return pl.pallas_call(
        body,
        out_shape=jax.ShapeDtypeStruct((M, D), jnp.bfloat16),
        in_specs=[pl.BlockSpec(memory_space=pltpu.VMEM)] * 4,
        out_specs=pl.BlockSpec(memory_space=pltpu.VMEM),
        scratch_shapes=[
            pltpu.VMEM((M, D), jnp.float32),
            pltpu.VMEM((_COMM_ROWS, D), jnp.bfloat16),
            pltpu.VMEM((512, D), jnp.bfloat16),
            pltpu.SemaphoreType.DMA((2,)),
            pltpu.SemaphoreType.DMA((4 * LOG_N,)),
        ],
        compiler_params=pltpu.CompilerParams(collective_id=0),
    )(x, Wg, Wu, Wd)


# device time: 73320 ns/iter; 1.7460x vs baseline; 1.0183x over previous
import jax
import jax.numpy as jnp
from jax import lax
from jax.experimental import pallas as pl
from jax.experimental.pallas import tpu as pltpu

N_DEV = 16
M = 1024
D = 1024
HC = D // 2
LOG_N = 4

_RS_DIMS_A = (1, 2, 4, 8)
_RS_DIMS_B = (4, 8, 1, 2)
_AG_DIMS_A = (8, 4, 2, 1)
_AG_DIMS_B = (2, 1, 8, 4)
_RS_OFF = (0, 512, 768, 896)
_COMM_ROWS = 960


def kernel(x, Wg, Wu, Wd):
    x = x.astype(jnp.bfloat16)
    Wg = Wg.astype(jnp.bfloat16)
    Wu = Wu.astype(jnp.bfloat16)
    Wd = Wd.astype(jnp.bfloat16)

    def body(x_ref, wg_ref, wu_ref, wd_ref, out_ref,
             acc_ref, comm_ref, stage_ref, send_sems, recv_sems):
        my = lax.axis_index("i")

        barrier_sem = pltpu.get_barrier_semaphore()
        for k in range(LOG_N):
            pl.semaphore_signal(
                barrier_sem, inc=1,
                device_id=(my ^ (1 << k),),
                device_id_type=pl.DeviceIdType.MESH,
            )
        pl.semaphore_wait(barrier_sem, LOG_N)

        def compute_half(start):
            xh = x_ref[pl.ds(start, 512), :]
            gate = jnp.dot(xh, wg_ref[:, :], preferred_element_type=jnp.float32)
            up = jnp.dot(xh, wu_ref[:, :], preferred_element_type=jnp.float32)
            h = (gate * (up * jax.nn.sigmoid(up))).astype(jnp.bfloat16)
            acc_ref[pl.ds(start, 512), :] = jnp.dot(
                h, wd_ref[:, :], preferred_element_type=jnp.float32
            )

        def rs_exchange(t, half, send_start, cbase, snd, sem_idx, partner):
            cols = pl.ds(cbase, HC)
            stage_ref[pl.ds(0, half), cols] = acc_ref[
                pl.ds(send_start, half), cols
            ].astype(jnp.bfloat16)
            rdma = pltpu.make_async_remote_copy(
                src_ref=stage_ref.at[pl.ds(0, half), cols],
                dst_ref=comm_ref.at[pl.ds(_RS_OFF[t], half), cols],
                send_sem=send_sems.at[snd],
                recv_sem=recv_sems.at[sem_idx],
                device_id=(partner,),
                device_id_type=pl.DeviceIdType.MESH,
            )
            rdma.start()
            return rdma

        def rs_add(t, half, keep, cbase):
            cols = pl.ds(cbase, HC)
            acc_ref[pl.ds(keep, half), cols] = (
                acc_ref[pl.ds(keep, half), cols]
                + comm_ref[pl.ds(_RS_OFF[t], half), cols].astype(jnp.float32)
            )

        dA, dB = _RS_DIMS_A[0], _RS_DIMS_B[0]
        upA, upB = (my & dA) != 0, (my & dB) != 0
        sendA = pl.multiple_of(jnp.where(upA, 0, 512), 64)
        keepA = pl.multiple_of(jnp.where(upA, 512, 0), 64)
        sendB = pl.multiple_of(jnp.where(upB, 0, 512), 64)
        keepB = pl.multiple_of(jnp.where(upB, 512, 0), 64)
        compute_half(sendA)
        rdma_a = rs_exchange(0, 512, sendA, 0, 0, 0, my ^ dA)
        xh = x_ref[pl.ds(keepA, 512), :]
        gate = jnp.dot(xh, wg_ref[:, :], preferred_element_type=jnp.float32)
        up = jnp.dot(xh, wu_ref[:, :], preferred_element_type=jnp.float32)
        h2 = (gate * (up * jax.nn.sigmoid(up))).astype(jnp.bfloat16)
        acc_ref[pl.ds(keepA, 512), pl.ds(HC, HC)] = jnp.dot(
            h2, wd_ref[:, HC:], preferred_element_type=jnp.float32
        )
        rdma_b = rs_exchange(0, 512, sendB, HC, 1, LOG_N, my ^ dB)
        acc_ref[pl.ds(keepA, 512), pl.ds(0, HC)] = jnp.dot(
            h2, wd_ref[:, :HC], preferred_element_type=jnp.float32
        )
        rdma_a.wait()
        rs_add(0, 512, keepA, 0)
        rdma_b.wait()
        rs_add(0, 512, keepB, HC)
        sA, sB = keepA, keepB

        for t in range(1, LOG_N):
            half = 512 >> t
            dA, dB = _RS_DIMS_A[t], _RS_DIMS_B[t]
            upA, upB = (my & dA) != 0, (my & dB) != 0
            sendA = pl.multiple_of(sA + jnp.where(upA, 0, half), 64)
            keepA = pl.multiple_of(sA + jnp.where(upA, half, 0), 64)
            sendB = pl.multiple_of(sB + jnp.where(upB, 0, half), 64)
            keepB = pl.multiple_of(sB + jnp.where(upB, half, 0), 64)
            rdma_a = rs_exchange(t, half, sendA, 0, 0, t, my ^ dA)
            rdma_b = rs_exchange(t, half, sendB, HC, 1, LOG_N + t, my ^ dB)
            rdma_a.wait()
            rs_add(t, half, keepA, 0)
            rdma_b.wait()
            rs_add(t, half, keepB, HC)
            sA, sB = keepA, keepB

        sA = pl.multiple_of(sA, 64)
        sB = pl.multiple_of(sB, 64)
        out_ref[pl.ds(sA, 64), pl.ds(0, HC)] = acc_ref[
            pl.ds(sA, 64), pl.ds(0, HC)
        ].astype(jnp.bfloat16)
        out_ref[pl.ds(sB, 64), pl.ds(HC, HC)] = acc_ref[
            pl.ds(sB, 64), pl.ds(HC, HC)
        ].astype(jnp.bfloat16)

        def ag_exchange(L, s, cbase, snd, sem_idx, partner):
            cols = pl.ds(cbase, HC)
            rdma = pltpu.make_async_remote_copy(
                src_ref=out_ref.at[pl.ds(s, L), cols],
                dst_ref=out_ref.at[pl.ds(s, L), cols],
                send_sem=send_sems.at[snd],
                recv_sem=recv_sems.at[sem_idx],
                device_id=(partner,),
                device_id_type=pl.DeviceIdType.MESH,
            )
            rdma.start()
            return rdma

        for t in range(LOG_N):
            L = 64 << t
            dA, dB = _AG_DIMS_A[t], _AG_DIMS_B[t]
            rdma_a = ag_exchange(L, sA, 0, 0, 2 * LOG_N + t, my ^ dA)
            rdma_b = ag_exchange(L, sB, HC, 1, 3 * LOG_N + t, my ^ dB)
            rdma_a.wait()
            rdma_b.wait()
            sA = pl.multiple_of(jnp.where((my & dA) != 0, sA - L, sA), 64)
            sB = pl.multiple_of(jnp.where((my & dB) != 0, sB - L, sB), 64)

    return pl.pallas_call(
        body,
        out_shape=jax.ShapeDtypeStruct((M, D), jnp.bfloat16),
        in_specs=[pl.BlockSpec(memory_space=pltpu.VMEM)] * 4,
        out_specs=pl.BlockSpec(memory_space=pltpu.VMEM),
        scratch_shapes=[
            pltpu.VMEM((M, D), jnp.float32),
            pltpu.VMEM((_COMM_ROWS, D), jnp.bfloat16),
            pltpu.VMEM((512, D), jnp.bfloat16),
            pltpu.SemaphoreType.DMA((2,)),
            pltpu.SemaphoreType.DMA((4 * LOG_N,)),
        ],
        compiler_params=pltpu.CompilerParams(collective_id=0),
    )(x, Wg, Wu, Wd)


# device time: 71306 ns/iter; 1.7953x vs baseline; 1.0282x over previous
import jax
import jax.numpy as jnp
from jax import lax
from jax.experimental import pallas as pl
from jax.experimental.pallas import tpu as pltpu

N_DEV = 16
M = 1024
D = 1024
NQ = 4
QC = D // NQ
LOG_N = 4

_ORDERS = ((1, 2, 4, 8), (2, 1, 8, 4), (4, 8, 1, 2), (8, 4, 2, 1))
_RS_OFF = (0, 512, 768, 896)
_COMM_ROWS = 960


def kernel(x, Wg, Wu, Wd):
    x = x.astype(jnp.bfloat16)
    Wg = Wg.astype(jnp.bfloat16)
    Wu = Wu.astype(jnp.bfloat16)
    Wd = Wd.astype(jnp.bfloat16)

    def body(x_ref, wg_ref, wu_ref, wd_ref, out_ref,
             acc_ref, comm_ref, stage_ref, send_sems, recv_sems):
        my = lax.axis_index("i")

        barrier_sem = pltpu.get_barrier_semaphore()
        for k in range(LOG_N):
            pl.semaphore_signal(
                barrier_sem, inc=1,
                device_id=(my ^ (1 << k),),
                device_id_type=pl.DeviceIdType.MESH,
            )
        pl.semaphore_wait(barrier_sem, LOG_N)

        def splits(s, t, d):
            half = 512 >> t
            upper = (my & d) != 0
            send = pl.multiple_of(s + jnp.where(upper, 0, half), 64)
            keep = pl.multiple_of(s + jnp.where(upper, half, 0), 64)
            return send, keep

        def rs_exchange(q, t, send_start):
            half = 512 >> t
            cols = pl.ds(q * QC, QC)
            stage_ref[pl.ds(0, half), cols] = acc_ref[
                pl.ds(send_start, half), cols
            ].astype(jnp.bfloat16)
            rdma = pltpu.make_async_remote_copy(
                src_ref=stage_ref.at[pl.ds(0, half), cols],
                dst_ref=comm_ref.at[pl.ds(_RS_OFF[t], half), cols],
                send_sem=send_sems.at[q],
                recv_sem=recv_sems.at[q * LOG_N + t],
                device_id=(my ^ _ORDERS[q][t],),
                device_id_type=pl.DeviceIdType.MESH,
            )
            rdma.start()
            return rdma

        def rs_add(q, t, keep):
            half = 512 >> t
            cols = pl.ds(q * QC, QC)
            acc_ref[pl.ds(keep, half), cols] = (
                acc_ref[pl.ds(keep, half), cols]
                + comm_ref[pl.ds(_RS_OFF[t], half), cols].astype(jnp.float32)
            )

        send0, keep0 = splits(my * 0, 0, _ORDERS[0][0])
        xh = x_ref[pl.ds(send0, 512), :]
        gate = jnp.dot(xh, wg_ref[:, :], preferred_element_type=jnp.float32)
        up = jnp.dot(xh, wu_ref[:, :], preferred_element_type=jnp.float32)
        h1 = (gate * (up * jax.nn.sigmoid(up))).astype(jnp.bfloat16)
        acc_ref[pl.ds(send0, 512), :] = jnp.dot(
            h1, wd_ref[:, :], preferred_element_type=jnp.float32
        )
        rdmas = [None] * NQ
        keeps = [None] * NQ
        rdmas[0] = rs_exchange(0, 0, send0)
        keeps[0] = keep0
        xh = x_ref[pl.ds(keep0, 512), :]
        gate = jnp.dot(xh, wg_ref[:, :], preferred_element_type=jnp.float32)
        up = jnp.dot(xh, wu_ref[:, :], preferred_element_type=jnp.float32)
        h2 = (gate * (up * jax.nn.sigmoid(up))).astype(jnp.bfloat16)
        for q in (1, 2, 3):
            cols = pl.ds(q * QC, QC)
            acc_ref[pl.ds(keep0, 512), cols] = jnp.dot(
                h2, wd_ref[:, q * QC:(q + 1) * QC],
                preferred_element_type=jnp.float32,
            )
            send_q, keeps[q] = splits(my * 0, 0, _ORDERS[q][0])
            rdmas[q] = rs_exchange(q, 0, send_q)
        acc_ref[pl.ds(keep0, 512), pl.ds(0, QC)] = jnp.dot(
            h2, wd_ref[:, 0:QC], preferred_element_type=jnp.float32
        )
        ss = [None] * NQ
        for q in range(NQ):
            rdmas[q].wait()
            rs_add(q, 0, keeps[q])
            ss[q] = keeps[q]

        for t in range(1, LOG_N):
            for q in range(NQ):
                send_q, keeps[q] = splits(ss[q], t, _ORDERS[q][t])
                rdmas[q] = rs_exchange(q, t, send_q)
            for q in range(NQ):
                rdmas[q].wait()
                rs_add(q, t, keeps[q])
                ss[q] = keeps[q]

        for q in range(NQ):
            cols = pl.ds(q * QC, QC)
            out_ref[pl.ds(ss[q], 64), cols] = acc_ref[
                pl.ds(ss[q], 64), cols
            ].astype(jnp.bfloat16)

        for t in range(LOG_N):
            L = 64 << t
            for q in range(NQ):
                d = _ORDERS[q][LOG_N - 1 - t]
                cols = pl.ds(q * QC, QC)
                s = pl.multiple_of(ss[q], 64)
                rdma = pltpu.make_async_remote_copy(
                    src_ref=out_ref.at[pl.ds(s, L), cols],
                    dst_ref=out_ref.at[pl.ds(s, L), cols],
                    send_sem=send_sems.at[q],
                    recv_sem=recv_sems.at[(LOG_N + t) * NQ + q],
                    device_id=(my ^ d,),
                    device_id_type=pl.DeviceIdType.MESH,
                )
                rdma.start()
                rdmas[q] = rdma
                keeps[q] = jnp.where((my & d) != 0, s - L, s)
            for q in range(NQ):
                rdmas[q].wait()
                ss[q] = keeps[q]

    return pl.pallas_call(
        body,
        out_shape=jax.ShapeDtypeStruct((M, D), jnp.bfloat16),
        in_specs=[pl.BlockSpec(memory_space=pltpu.VMEM)] * 4,
        out_specs=pl.BlockSpec(memory_space=pltpu.VMEM),
        scratch_shapes=[
            pltpu.VMEM((M, D), jnp.float32),
            pltpu.VMEM((_COMM_ROWS, D), jnp.bfloat16),
            pltpu.VMEM((512, D), jnp.bfloat16),
            pltpu.SemaphoreType.DMA((NQ,)),
            pltpu.SemaphoreType.DMA((8 * NQ,)),
        ],
        compiler_params=pltpu.CompilerParams(collective_id=0),
    )(x, Wg, Wu, Wd)


# device time: 66340 ns/iter; 1.9297x vs baseline; 1.0749x over previous
import jax
import jax.numpy as jnp
from jax import lax
from jax.experimental import pallas as pl
from jax.experimental.pallas import tpu as pltpu

N_DEV = 16
M = 1024
D = 1024
NQ = 4
QC = D // NQ
LOG_N = 4

_ORDERS = ((1, 2, 4, 8), (2, 1, 8, 4), (4, 8, 1, 2), (8, 4, 2, 1))
_RS_OFF = (0, 512, 768, 896)
_COMM_ROWS = 960


def kernel(x, Wg, Wu, Wd):
    x = x.astype(jnp.bfloat16)
    Wg = Wg.astype(jnp.bfloat16)
    Wu = Wu.astype(jnp.bfloat16)
    Wd = Wd.astype(jnp.bfloat16)

    def body(x_ref, wg_ref, wu_ref, wd_ref, out_ref,
             acc_ref, comm_ref, stage_ref, send_sems, recv_sems):
        my = lax.axis_index("i")

        barrier_sem = pltpu.get_barrier_semaphore()
        for k in range(LOG_N):
            pl.semaphore_signal(
                barrier_sem, inc=1,
                device_id=(my ^ (1 << k),),
                device_id_type=pl.DeviceIdType.MESH,
            )
        pl.semaphore_wait(barrier_sem, LOG_N)

        def splits(s, t, d):
            half = 512 >> t
            upper = (my & d) != 0
            send = pl.multiple_of(s + jnp.where(upper, 0, half), 64)
            keep = pl.multiple_of(s + jnp.where(upper, half, 0), 64)
            return send, keep

        def rs_exchange(q, t, send_start):
            half = 512 >> t
            cols = pl.ds(q * QC, QC)
            stage_ref[pl.ds(0, half), cols] = acc_ref[
                pl.ds(send_start, half), cols
            ].astype(jnp.bfloat16)
            rdma = pltpu.make_async_remote_copy(
                src_ref=stage_ref.at[pl.ds(0, half), cols],
                dst_ref=comm_ref.at[pl.ds(_RS_OFF[t], half), cols],
                send_sem=send_sems.at[q],
                recv_sem=recv_sems.at[q * LOG_N + t],
                device_id=(my ^ _ORDERS[q][t],),
                device_id_type=pl.DeviceIdType.MESH,
            )
            rdma.start()
            return rdma

        def rs_add(q, t, keep):
            half = 512 >> t
            cols = pl.ds(q * QC, QC)
            acc_ref[pl.ds(keep, half), cols] = (
                acc_ref[pl.ds(keep, half), cols]
                + comm_ref[pl.ds(_RS_OFF[t], half), cols].astype(jnp.float32)
            )

        send0, keep0 = splits(my * 0, 0, _ORDERS[0][0])
        xh = x_ref[pl.ds(send0, 512), :]
        gate = jnp.dot(xh, wg_ref[:, :], preferred_element_type=jnp.float32)
        up = jnp.dot(xh, wu_ref[:, :], preferred_element_type=jnp.float32)
        h1 = (gate * (up * jax.nn.sigmoid(up))).astype(jnp.bfloat16)
        acc_ref[pl.ds(send0, 512), :] = jnp.dot(
            h1, wd_ref[:, :], preferred_element_type=jnp.float32
        )
        rdmas = [None] * NQ
        keeps = [None] * NQ
        rdmas[0] = rs_exchange(0, 0, send0)
        keeps[0] = keep0
        xh = x_ref[pl.ds(keep0, 512), :]
        gate = jnp.dot(xh, wg_ref[:, :], preferred_element_type=jnp.float32)
        up = jnp.dot(xh, wu_ref[:, :], preferred_element_type=jnp.float32)
        h2 = (gate * (up * jax.nn.sigmoid(up))).astype(jnp.bfloat16)
        for q in (1, 2, 3):
            cols = pl.ds(q * QC, QC)
            acc_ref[pl.ds(keep0, 512), cols] = jnp.dot(
                h2, wd_ref[:, q * QC:(q + 1) * QC],
                preferred_element_type=jnp.float32,
            )
            send_q, keeps[q] = splits(my * 0, 0, _ORDERS[q][0])
            rdmas[q] = rs_exchange(q, 0, send_q)
        acc_ref[pl.ds(keep0, 512), pl.ds(0, QC)] = jnp.dot(
            h2, wd_ref[:, 0:QC], preferred_element_type=jnp.float32
        )
        def ag_exchange(q, t, s):
            L = 64 << t
            d = _ORDERS[q][LOG_N - 1 - t]
            cols = pl.ds(q * QC, QC)
            s = pl.multiple_of(s, 64)
            rdma = pltpu.make_async_remote_copy(
                src_ref=out_ref.at[pl.ds(s, L), cols],
                dst_ref=out_ref.at[pl.ds(s, L), cols],
                send_sem=send_sems.at[q],
                recv_sem=recv_sems.at[(LOG_N + t) * NQ + q],
                device_id=(my ^ d,),
                device_id_type=pl.DeviceIdType.MESH,
            )
            rdma.start()
            return rdma, jnp.where((my & d) != 0, s - L, s)

        ss = [None] * NQ
        for t in range(1, LOG_N + 1):
            for q in range(NQ):
                rdmas[q].wait()
                rs_add(q, t - 1, keeps[q])
                ss[q] = keeps[q]
                if t < LOG_N:
                    send_q, keeps[q] = splits(ss[q], t, _ORDERS[q][t])
                    rdmas[q] = rs_exchange(q, t, send_q)
                else:
                    cols = pl.ds(q * QC, QC)
                    out_ref[pl.ds(ss[q], 64), cols] = acc_ref[
                        pl.ds(ss[q], 64), cols
                    ].astype(jnp.bfloat16)
                    rdmas[q], keeps[q] = ag_exchange(q, 0, ss[q])

        for t in range(1, LOG_N + 1):
            for q in range(NQ):
                rdmas[q].wait()
                ss[q] = keeps[q]
                if t < LOG_N:
                    rdmas[q], keeps[q] = ag_exchange(q, t, ss[q])

    return pl.pallas_call(
        body,
        out_shape=jax.ShapeDtypeStruct((M, D), jnp.bfloat16),
        in_specs=[pl.BlockSpec(memory_space=pltpu.VMEM)] * 4,
        out_specs=pl.BlockSpec(memory_space=pltpu.VMEM),
        scratch_shapes=[
            pltpu.VMEM((M, D), jnp.float32),
            pltpu.VMEM((_COMM_ROWS, D), jnp.bfloat16),
            pltpu.VMEM((512, D), jnp.bfloat16),
            pltpu.SemaphoreType.DMA((NQ,)),
            pltpu.SemaphoreType.DMA((8 * NQ,)),
        ],
        compiler_params=pltpu.CompilerParams(collective_id=0),
    )(x, Wg, Wu, Wd)


# device time: 63028 ns/iter; 2.0311x vs baseline; 1.0525x over previous
import jax
import jax.numpy as jnp
from jax import lax
from jax.experimental import pallas as pl
from jax.experimental.pallas import tpu as pltpu

N_DEV = 16
M = 1024
D = 1024
NQ = 4
QC = D // NQ
LOG_N = 4

_ORDERS = ((8, 4, 2, 1), (4, 8, 1, 2), (2, 1, 8, 4), (1, 2, 4, 8))
_RS_OFF = (0, 512, 768, 896)
_COMM_ROWS = 960


def kernel(x, Wg, Wu, Wd):
    x = x.astype(jnp.bfloat16)
    Wg = Wg.astype(jnp.bfloat16)
    Wu = Wu.astype(jnp.bfloat16)
    Wd = Wd.astype(jnp.bfloat16)

    def body(x_ref, wg_ref, wu_ref, wd_ref, out_ref,
             acc_ref, comm_ref, stage_ref, send_sems, recv_sems):
        my = lax.axis_index("i")

        barrier_sem = pltpu.get_barrier_semaphore()
        for k in range(LOG_N):
            pl.semaphore_signal(
                barrier_sem, inc=1,
                device_id=(my ^ (1 << k),),
                device_id_type=pl.DeviceIdType.MESH,
            )
        pl.semaphore_wait(barrier_sem, LOG_N)

        def splits(s, t, d):
            half = 512 >> t
            upper = (my & d) != 0
            send = pl.multiple_of(s + jnp.where(upper, 0, half), 64)
            keep = pl.multiple_of(s + jnp.where(upper, half, 0), 64)
            return send, keep

        def rs_exchange(q, t, send_start):
            half = 512 >> t
            cols = pl.ds(q * QC, QC)
            stage_ref[pl.ds(0, half), cols] = acc_ref[
                pl.ds(send_start, half), cols
            ].astype(jnp.bfloat16)
            rdma = pltpu.make_async_remote_copy(
                src_ref=stage_ref.at[pl.ds(0, half), cols],
                dst_ref=comm_ref.at[pl.ds(_RS_OFF[t], half), cols],
                send_sem=send_sems.at[q],
                recv_sem=recv_sems.at[q * LOG_N + t],
                device_id=(my ^ _ORDERS[q][t],),
                device_id_type=pl.DeviceIdType.MESH,
            )
            rdma.start()
            return rdma

        def rs_add(q, t, keep):
            half = 512 >> t
            cols = pl.ds(q * QC, QC)
            acc_ref[pl.ds(keep, half), cols] = (
                acc_ref[pl.ds(keep, half), cols]
                + comm_ref[pl.ds(_RS_OFF[t], half), cols].astype(jnp.float32)
            )

        send0, keep0 = splits(my * 0, 0, _ORDERS[0][0])
        xh = x_ref[pl.ds(send0, 512), :]
        gate = jnp.dot(xh, wg_ref[:, :], preferred_element_type=jnp.float32)
        up = jnp.dot(xh, wu_ref[:, :], preferred_element_type=jnp.float32)
        h1 = (gate * (up * jax.nn.sigmoid(up))).astype(jnp.bfloat16)
        rdmas = [None] * NQ
        keeps = [None] * NQ
        acc_ref[pl.ds(send0, 512), pl.ds(0, QC)] = jnp.dot(
            h1, wd_ref[:, 0:QC], preferred_element_type=jnp.float32
        )
        rdmas[0] = rs_exchange(0, 0, send0)
        keeps[0] = keep0
        acc_ref[pl.ds(send0, 512), pl.ds(QC, D - QC)] = jnp.dot(
            h1, wd_ref[:, QC:], preferred_element_type=jnp.float32
        )
        xh = x_ref[pl.ds(keep0, 512), :]
        gate = jnp.dot(xh, wg_ref[:, :], preferred_element_type=jnp.float32)
        up = jnp.dot(xh, wu_ref[:, :], preferred_element_type=jnp.float32)
        h2 = (gate * (up * jax.nn.sigmoid(up))).astype(jnp.bfloat16)
        for q in (1, 2, 3):
            cols = pl.ds(q * QC, QC)
            acc_ref[pl.ds(keep0, 512), cols] = jnp.dot(
                h2, wd_ref[:, q * QC:(q + 1) * QC],
                preferred_element_type=jnp.float32,
            )
            send_q, keeps[q] = splits(my * 0, 0, _ORDERS[q][0])
            rdmas[q] = rs_exchange(q, 0, send_q)
        acc_ref[pl.ds(keep0, 512), pl.ds(0, QC)] = jnp.dot(
            h2, wd_ref[:, 0:QC], preferred_element_type=jnp.float32
        )
        def ag_exchange(q, t, s):
            L = 64 << t
            d = _ORDERS[q][LOG_N - 1 - t]
            cols = pl.ds(q * QC, QC)
            s = pl.multiple_of(s, 64)
            rdma = pltpu.make_async_remote_copy(
                src_ref=out_ref.at[pl.ds(s, L), cols],
                dst_ref=out_ref.at[pl.ds(s, L), cols],
                send_sem=send_sems.at[q],
                recv_sem=recv_sems.at[(LOG_N + t) * NQ + q],
                device_id=(my ^ d,),
                device_id_type=pl.DeviceIdType.MESH,
            )
            rdma.start()
            return rdma, jnp.where((my & d) != 0, s - L, s)

        ss = [None] * NQ
        for t in range(1, LOG_N + 1):
            for q in range(NQ):
                rdmas[q].wait()
                rs_add(q, t - 1, keeps[q])
                ss[q] = keeps[q]
                if t < LOG_N:
                    send_q, keeps[q] = splits(ss[q], t, _ORDERS[q][t])
                    rdmas[q] = rs_exchange(q, t, send_q)
                else:
                    cols = pl.ds(q * QC, QC)
                    out_ref[pl.ds(ss[q], 64), cols] = acc_ref[
                        pl.ds(ss[q], 64), cols
                    ].astype(jnp.bfloat16)
                    rdmas[q], keeps[q] = ag_exchange(q, 0, ss[q])

        for t in range(1, LOG_N + 1):
            for q in range(NQ):
                rdmas[q].wait()
                ss[q] = keeps[q]
                if t < LOG_N:
                    rdmas[q], keeps[q] = ag_exchange(q, t, ss[q])

    return pl.pallas_call(
        body,
        out_shape=jax.ShapeDtypeStruct((M, D), jnp.bfloat16),
        in_specs=[pl.BlockSpec(memory_space=pltpu.VMEM)] * 4,
        out_specs=pl.BlockSpec(memory_space=pltpu.VMEM),
        scratch_shapes=[
            pltpu.VMEM((M, D), jnp.float32),
            pltpu.VMEM((_COMM_ROWS, D), jnp.bfloat16),
            pltpu.VMEM((512, D), jnp.bfloat16),
            pltpu.SemaphoreType.DMA((NQ,)),
            pltpu.SemaphoreType.DMA((8 * NQ,)),
        ],
        compiler_params=pltpu.CompilerParams(collective_id=0),
    )(x, Wg, Wu, Wd)
